# Initial kernel scaffold; baseline (speedup 1.0000x reference)
import jax
import jax.numpy as jnp
from jax.experimental import pallas as pl


def kernel(x, lap_indices, lap_values, weight, bias):
    raise NotImplementedError("write your pallas kernel here")


# speedup vs baseline: 4.2701x; 4.2701x over previous
import functools

import jax
import jax.numpy as jnp
from jax import lax
from jax.experimental import pallas as pl
from jax.experimental.pallas import tpu as pltpu
from jax.experimental.pallas import tpu_sc as plsc

V = 10000
E = 320000
F = 128
NC = 2
NS = 16
VP = 10240
ET = E // (NC * NS)
K = 80
NCHUNK = ET // K
RT = VP // NS
ZROWS = 64


def _spmm_sc_partial():
    mesh = plsc.VectorSubcoreMesh(core_axis_name="c", subcore_axis_name="s")

    @functools.partial(
        pl.kernel,
        out_type=jax.ShapeDtypeStruct((NC, VP, F), jnp.float32),
        mesh=mesh,
        scratch_types=[
            pltpu.VMEM((K,), jnp.int32),
            pltpu.VMEM((K,), jnp.int32),
            pltpu.VMEM((ET,), jnp.float32),
            pltpu.VMEM((K, F), jnp.float32),
            pltpu.VMEM((ZROWS, F), jnp.float32),
            pltpu.VMEM_SHARED((VP, F), jnp.float32),
            pltpu.SemaphoreType.DMA,
        ],
    )
    def spmm(x_hbm, cols_hbm, rows_hbm, vals_hbm, out_hbm,
             cbuf, rbuf, vv, gbuf, zbuf, acc, sem):
        cid = lax.axis_index("c")
        sid = lax.axis_index("s")
        eb = (cid * NS + sid) * ET

        zv = jnp.zeros((16,), jnp.float32)

        def zero_body(i, _):
            for f in range(F // 16):
                zbuf[i, pl.ds(16 * f, 16)] = zv
            return 0

        lax.fori_loop(0, ZROWS, zero_body, 0)
        for t in range(RT // ZROWS):
            pltpu.sync_copy(zbuf, acc.at[pl.ds(sid * RT + t * ZROWS, ZROWS)])

        pltpu.sync_copy(vals_hbm.at[pl.ds(eb, ET)], vv)

        plsc.subcore_barrier()

        def chunk_body(k, _):
            base = eb + k * K
            pltpu.sync_copy(cols_hbm.at[pl.ds(base, K)], cbuf)
            pltpu.sync_copy(rows_hbm.at[pl.ds(base, K)], rbuf)
            pltpu.async_copy(x_hbm.at[cbuf], gbuf, sem).wait()

            def scale_body(g, _):
                vvec = vv[pl.ds(k * K + g * 16, 16)]
                for jj in range(16):
                    vval = vvec[jj]
                    j = g * 16 + jj
                    for f in range(F // 16):
                        gf = gbuf[j, pl.ds(16 * f, 16)]
                        gbuf[j, pl.ds(16 * f, 16)] = gf * vval
                return 0

            lax.fori_loop(0, K // 16, scale_body, 0)
            pltpu.sync_copy(gbuf, acc.at[rbuf], add=True)
            return 0

        lax.fori_loop(0, NCHUNK, chunk_body, 0)

        plsc.subcore_barrier()

        pltpu.sync_copy(acc.at[pl.ds(sid * RT, RT)],
                        out_hbm.at[cid, pl.ds(sid * RT, RT)])

    return spmm


_spmm = _spmm_sc_partial()


def _merge_body(p_ref, o_ref):
    o_ref[...] = p_ref[0] + p_ref[1]


def _merge_partials(p):
    grid = 8
    vb = VP // grid
    return pl.pallas_call(
        _merge_body,
        grid=(grid,),
        in_specs=[pl.BlockSpec((NC, vb, F), lambda i: (0, i, 0))],
        out_specs=pl.BlockSpec((vb, F), lambda i: (i, 0)),
        out_shape=jax.ShapeDtypeStruct((VP, F), jnp.float32),
    )(p)


def _final_body(x0_ref, x1_ref, q_ref, w_ref, b_ref, o_ref):
    x0b = x0_ref[...]
    x1b = x1_ref[...]
    x2b = 2.0 * (q_ref[0] + q_ref[1]) - x0b
    dims = (((0,), (1,)), ((), ()))
    acc = lax.dot_general(w_ref[0], x0b, dims,
                          preferred_element_type=jnp.float32)
    acc += lax.dot_general(w_ref[1], x1b, dims,
                           preferred_element_type=jnp.float32)
    acc += lax.dot_general(w_ref[2], x2b, dims,
                           preferred_element_type=jnp.float32)
    o_ref[...] = acc + b_ref[...]


def _final_combine(x0, x1, q, weight, bias_col):
    grid = 8
    vb = VP // grid
    return pl.pallas_call(
        _final_body,
        grid=(grid,),
        in_specs=[
            pl.BlockSpec((vb, F), lambda i: (i, 0)),
            pl.BlockSpec((vb, F), lambda i: (i, 0)),
            pl.BlockSpec((NC, vb, F), lambda i: (0, i, 0)),
            pl.BlockSpec((3, F, F), lambda i: (0, 0, 0)),
            pl.BlockSpec((F, 1), lambda i: (0, 0)),
        ],
        out_specs=pl.BlockSpec((F, vb), lambda i: (0, i)),
        out_shape=jax.ShapeDtypeStruct((F, VP), jnp.float32),
    )(x0, x1, q, weight, bias_col)


def kernel(x, lap_indices, lap_values, weight, bias):
    b, cin, v = x.shape
    x0 = jnp.transpose(x[0]).astype(jnp.float32)
    x0 = jnp.pad(x0, ((0, VP - V), (0, 0)))
    rows = lap_indices[0].astype(jnp.int32)
    cols = lap_indices[1].astype(jnp.int32)
    vals = lap_values.astype(jnp.float32)

    p = _spmm(x0, cols, rows, vals)
    x1 = _merge_partials(p)
    q = _spmm(x1, cols, rows, vals)
    bias_col = bias.reshape(F, 1).astype(jnp.float32)
    out = _final_combine(x0, x1, q, weight, bias_col)
    return out[:, :V].reshape(1, F, V)


# speedup vs baseline: 10.5990x; 2.4822x over previous
import functools

import jax
import jax.numpy as jnp
from jax import lax
from jax.experimental import pallas as pl
from jax.experimental.pallas import tpu as pltpu
from jax.experimental.pallas import tpu_sc as plsc

V = 10000
E = 320000
F = 128
NC = 2
NS = 16
VP = 10240
ET = E // (NC * NS)
K = 80
NCHUNK = ET // K
NG = 3
RT = VP // NS


def _spmm_sc_partial():
    mesh = plsc.VectorSubcoreMesh(core_axis_name="c", subcore_axis_name="s")

    @functools.partial(
        pl.kernel,
        out_type=jax.ShapeDtypeStruct((NC, VP, F), jnp.float32),
        mesh=mesh,
        scratch_types=[
            pltpu.VMEM((ET,), jnp.int32),
        ] + [pltpu.VMEM((K, F), jnp.float32) for _ in range(NG)]
          + [pltpu.VMEM((K,), jnp.int32) for _ in range(NG)]
          + [pltpu.VMEM((K,), jnp.float32) for _ in range(NG)]
          + [
            pltpu.VMEM_SHARED((VP, F), jnp.float32),
            pltpu.SemaphoreType.DMA,
            pltpu.SemaphoreType.DMA,
        ],
    )
    def spmm(x_hbm, cols_hbm, rows_hbm, vals_hbm, zeros_hbm, out_hbm,
             cbig, g0, g1, g2, r0, r1, r2, v0, v1, v2,
             acc, gsem, isem):
        gb = [g0, g1, g2]
        rb = [r0, r1, r2]
        vb = [v0, v1, v2]
        cid = lax.axis_index("c")
        sid = lax.axis_index("s")
        tile = cid * NS + sid
        eb = tile * ET

        pltpu.sync_copy(cols_hbm.at[pl.ds(eb, ET)], cbig)
        pltpu.sync_copy(zeros_hbm, acc.at[pl.ds(sid * RT, RT)])

        plsc.subcore_barrier()

        def issue(c, b):
            pltpu.async_copy(rows_hbm.at[pl.ds(eb + c * K, K)], rb[b], isem)
            pltpu.async_copy(vals_hbm.at[pl.ds(eb + c * K, K)], vb[b], isem)
            pltpu.async_copy(x_hbm.at[cbig.at[pl.ds(c * K, K)]], gb[b], gsem)

        def process(c, b):
            pltpu.make_async_copy(rows_hbm.at[pl.ds(eb, K)], rb[b],
                                  isem).wait()
            pltpu.make_async_copy(vals_hbm.at[pl.ds(eb, K)], vb[b],
                                  isem).wait()
            pltpu.make_async_copy(x_hbm.at[cbig.at[pl.ds(c * K, K)]],
                                  gb[b], gsem).wait()
            gbuf = gb[b]
            vbuf = vb[b]

            def scale_body(g, _):
                vvec = vbuf[pl.ds(g * 16, 16)]
                for jj in range(16):
                    vval = vvec[jj]
                    j = g * 16 + jj
                    for f in range(F // 16):
                        gf = gbuf[j, pl.ds(16 * f, 16)]
                        gbuf[j, pl.ds(16 * f, 16)] = gf * vval
                return 0

            lax.fori_loop(0, K // 16, scale_body, 0)
            pltpu.sync_copy(gb[b], acc.at[rb[b]], add=True)

        for b in range(NG):
            issue(b, b)

        nmain = (NCHUNK // NG) * NG

        def loop_body(kk, _):
            for b in range(NG):
                c = kk * NG + b
                process(c, b)
                nxt = c + NG

                @pl.when(nxt < NCHUNK)
                def _():
                    issue(nxt, b)
            return 0

        lax.fori_loop(0, nmain // NG, loop_body, 0)
        for c in range(nmain, NCHUNK):
            process(c, c % NG)

        plsc.subcore_barrier()

        pltpu.sync_copy(acc.at[pl.ds(sid * RT, RT)],
                        out_hbm.at[cid, pl.ds(sid * RT, RT)])

    return spmm


_spmm = _spmm_sc_partial()


def _merge_body(p_ref, o_ref):
    o_ref[...] = p_ref[0] + p_ref[1]


def _merge_partials(p):
    grid = 8
    vb = VP // grid
    return pl.pallas_call(
        _merge_body,
        grid=(grid,),
        in_specs=[pl.BlockSpec((NC, vb, F), lambda i: (0, i, 0))],
        out_specs=pl.BlockSpec((vb, F), lambda i: (i, 0)),
        out_shape=jax.ShapeDtypeStruct((VP, F), jnp.float32),
    )(p)


def _final_body(x0_ref, x1_ref, q_ref, w_ref, b_ref, o_ref):
    x0b = x0_ref[...]
    x1b = x1_ref[...]
    x2b = 2.0 * (q_ref[0] + q_ref[1]) - x0b
    dims = (((0,), (1,)), ((), ()))
    acc = lax.dot_general(w_ref[0], x0b, dims,
                          preferred_element_type=jnp.float32)
    acc += lax.dot_general(w_ref[1], x1b, dims,
                           preferred_element_type=jnp.float32)
    acc += lax.dot_general(w_ref[2], x2b, dims,
                           preferred_element_type=jnp.float32)
    o_ref[...] = acc + b_ref[...]


def _final_combine(x0, x1, q, weight, bias_col):
    grid = 8
    vb = VP // grid
    return pl.pallas_call(
        _final_body,
        grid=(grid,),
        in_specs=[
            pl.BlockSpec((vb, F), lambda i: (i, 0)),
            pl.BlockSpec((vb, F), lambda i: (i, 0)),
            pl.BlockSpec((NC, vb, F), lambda i: (0, i, 0)),
            pl.BlockSpec((3, F, F), lambda i: (0, 0, 0)),
            pl.BlockSpec((F, 1), lambda i: (0, 0)),
        ],
        out_specs=pl.BlockSpec((F, vb), lambda i: (0, i)),
        out_shape=jax.ShapeDtypeStruct((F, VP), jnp.float32),
    )(x0, x1, q, weight, bias_col)


def kernel(x, lap_indices, lap_values, weight, bias):
    b, cin, v = x.shape
    x0 = jnp.transpose(x[0]).astype(jnp.float32)
    x0 = jnp.pad(x0, ((0, VP - V), (0, 0)))
    rows = lap_indices[0].astype(jnp.int32)
    cols = lap_indices[1].astype(jnp.int32)
    vals = lap_values.astype(jnp.float32)

    zb = jnp.zeros((RT, F), jnp.float32)
    p = _spmm(x0, cols, rows, vals, zb)
    x1 = _merge_partials(p)
    q = _spmm(x1, cols, rows, vals, zb)
    bias_col = bias.reshape(F, 1).astype(jnp.float32)
    out = _final_combine(x0, x1, q, weight, bias_col)
    return out[:, :V].reshape(1, F, V)


# speedup vs baseline: 10.8830x; 1.0268x over previous
import functools

import jax
import jax.numpy as jnp
from jax import lax
from jax.experimental import pallas as pl
from jax.experimental.pallas import tpu as pltpu
from jax.experimental.pallas import tpu_sc as plsc

V = 10000
E = 320000
F = 128
FH = F // 2
NC = 2
NS = 16
VP = 10240
ET = E // NS
K = 80
NCHUNK = ET // K
NG = 4
NSB = 2
RT = VP // NS


def _spmm_sc_half():
    mesh = plsc.VectorSubcoreMesh(core_axis_name="c", subcore_axis_name="s")

    @functools.partial(
        pl.kernel,
        out_type=[jax.ShapeDtypeStruct((VP, FH), jnp.float32),
                  jax.ShapeDtypeStruct((VP, FH), jnp.float32)],
        mesh=mesh,
        compiler_params=pltpu.CompilerParams(use_tc_tiling_on_sc=False),
        scratch_types=[
            pltpu.VMEM((ET,), jnp.int32),
        ] + [pltpu.VMEM((K, FH), jnp.float32) for _ in range(NG)]
          + [pltpu.VMEM((K, FH), jnp.float32) for _ in range(NSB)]
          + [pltpu.VMEM((K,), jnp.int32) for _ in range(NG)]
          + [pltpu.VMEM((K,), jnp.float32) for _ in range(NG)]
          + [pltpu.VMEM((K,), jnp.int32) for _ in range(NSB)]
          + [
            pltpu.VMEM_SHARED((VP, FH), jnp.float32),
            pltpu.SemaphoreType.DMA,
            pltpu.SemaphoreType.DMA,
            pltpu.SemaphoreType.DMA,
        ],
    )
    def spmm(xlo_hbm, xhi_hbm, cols_hbm, rows_hbm, vals_hbm, zeros_hbm,
             ylo_hbm, yhi_hbm,
             cbig, g0, g1, g2, g3, s0, s1, r0, r1, r2, r3,
             v0, v1, v2, v3, q0, q1, acc, gsem, isem, ssem):
        gb = [g0, g1, g2, g3]
        sb = [s0, s1]
        rb = [r0, r1, r2, r3]
        vb = [v0, v1, v2, v3]
        rs = [q0, q1]
        cid = lax.axis_index("c")
        sid = lax.axis_index("s")
        eb = sid * ET

        pltpu.sync_copy(cols_hbm.at[pl.ds(eb, ET)], cbig)
        pltpu.sync_copy(zeros_hbm, acc.at[pl.ds(sid * RT, RT)])

        plsc.subcore_barrier()

        def issue(c, b):
            pltpu.async_copy(rows_hbm.at[pl.ds(eb + c * K, K)], rb[b], isem)
            pltpu.async_copy(vals_hbm.at[pl.ds(eb + c * K, K)], vb[b], isem)
            idx = cbig.at[pl.ds(c * K, K)]

            @pl.when(cid == 0)
            def _():
                pltpu.async_copy(xlo_hbm.at[idx], gb[b], gsem)

            @pl.when(cid == 1)
            def _():
                pltpu.async_copy(xhi_hbm.at[idx], gb[b], gsem)

        def process(c, b, drain):
            b2 = b % NSB
            pltpu.make_async_copy(rows_hbm.at[pl.ds(eb, K)], rb[b],
                                  isem).wait()
            pltpu.make_async_copy(vals_hbm.at[pl.ds(eb, K)], vb[b],
                                  isem).wait()
            pltpu.make_async_copy(xlo_hbm.at[cbig.at[pl.ds(c * K, K)]],
                                  gb[b], gsem).wait()
            if drain:
                pltpu.make_async_copy(xlo_hbm.at[pl.ds(0, K)], sb[b2],
                                      ssem).wait()
            for g in range(K // 16):
                rs[b2][pl.ds(16 * g, 16)] = rb[b][pl.ds(16 * g, 16)]
            gbuf = gb[b]
            sbuf = sb[b2]
            vbuf = vb[b]

            def scale_body(g, _):
                vvec = vbuf[pl.ds(g * 16, 16)]
                for jj in range(16):
                    vval = vvec[jj]
                    j = g * 16 + jj
                    for f in range(FH // 16):
                        gf = gbuf[j, pl.ds(16 * f, 16)]
                        sbuf[j, pl.ds(16 * f, 16)] = gf * vval
                return 0

            lax.fori_loop(0, K // 16, scale_body, 0)
            pltpu.async_copy(sb[b2], acc.at[rs[b2]], ssem, add=True)

        for b in range(NG):
            issue(b, b)
        for b in range(NG):
            process(b, b, drain=(b >= NSB))
            issue(b + NG, b)

        nmain = (NCHUNK // NG) * NG

        def loop_body(kk, _):
            for b in range(NG):
                c = kk * NG + b
                process(c, b, drain=True)
                nxt = c + NG

                @pl.when(nxt < NCHUNK)
                def _():
                    issue(nxt, b)
            return 0

        lax.fori_loop(1, nmain // NG, loop_body, 0)
        for c in range(nmain, NCHUNK):
            process(c, c % NG, drain=True)
        for b2 in range(NSB):
            pltpu.make_async_copy(xlo_hbm.at[pl.ds(0, K)], sb[b2],
                                  ssem).wait()

        plsc.subcore_barrier()

        sl = pl.ds(sid * RT, RT)

        @pl.when(cid == 0)
        def _():
            pltpu.sync_copy(acc.at[sl], ylo_hbm.at[sl])

        @pl.when(cid == 1)
        def _():
            pltpu.sync_copy(acc.at[sl], yhi_hbm.at[sl])

    return spmm


_spmm = _spmm_sc_half()


def _final_body(x0_ref, x1l_ref, x1h_ref, y2l_ref, y2h_ref, w_ref, b_ref,
                o_ref):
    x0b = x0_ref[...]
    x1b = jnp.concatenate([x1l_ref[...], x1h_ref[...]], axis=1)
    y2b = jnp.concatenate([y2l_ref[...], y2h_ref[...]], axis=1)
    x2b = 2.0 * y2b - x0b
    dims = (((0,), (1,)), ((), ()))
    acc = lax.dot_general(w_ref[0], x0b, dims,
                          preferred_element_type=jnp.float32)
    acc += lax.dot_general(w_ref[1], x1b, dims,
                           preferred_element_type=jnp.float32)
    acc += lax.dot_general(w_ref[2], x2b, dims,
                           preferred_element_type=jnp.float32)
    o_ref[...] = acc + b_ref[...]


def _final_combine(x0, x1l, x1h, y2l, y2h, weight, bias_col):
    grid = 8
    vb = VP // grid
    half = pl.BlockSpec((vb, FH), lambda i: (i, 0))
    return pl.pallas_call(
        _final_body,
        grid=(grid,),
        in_specs=[
            pl.BlockSpec((vb, F), lambda i: (i, 0)),
            half, half, half, half,
            pl.BlockSpec((3, F, F), lambda i: (0, 0, 0)),
            pl.BlockSpec((F, 1), lambda i: (0, 0)),
        ],
        out_specs=pl.BlockSpec((F, vb), lambda i: (0, i)),
        out_shape=jax.ShapeDtypeStruct((F, VP), jnp.float32),
    )(x0, x1l, x1h, y2l, y2h, weight, bias_col)


def kernel(x, lap_indices, lap_values, weight, bias):
    b, cin, v = x.shape
    x0 = jnp.transpose(x[0]).astype(jnp.float32)
    x0 = jnp.pad(x0, ((0, VP - V), (0, 0)))
    xlo = x0[:, :FH]
    xhi = x0[:, FH:]
    rows = lap_indices[0].astype(jnp.int32)
    cols = lap_indices[1].astype(jnp.int32)
    vals = lap_values.astype(jnp.float32)

    zb = jnp.zeros((RT, FH), jnp.float32)
    x1l, x1h = _spmm(xlo, xhi, cols, rows, vals, zb)
    y2l, y2h = _spmm(x1l, x1h, cols, rows, vals, zb)
    bias_col = bias.reshape(F, 1).astype(jnp.float32)
    out = _final_combine(x0, x1l, x1h, y2l, y2h, weight, bias_col)
    return out[:, :V].reshape(1, F, V)
